# baseline (speedup 1.0000x reference)
import functools
import jax
import jax.numpy as jnp
from jax import lax
from jax.experimental import pallas as pl
from jax.experimental.pallas import tpu as pltpu
from jax.experimental.pallas import tpu_sc as plsc

_G = 256
_C = 80
_NW = 32


def _sc_body(x_hbm, deg_hbm, tab_hbm, tok_hbm, out_hbm,
             idx_v, rows_v, x_v, tok_v, tile_v, sem):
    cid = lax.axis_index("c")
    sid = lax.axis_index("s")
    wid = sid * 2 + cid
    n, d = x_hbm.shape
    n_chunks = n // _C

    @pl.when(wid == 0)
    def _():
        pltpu.sync_copy(tok_hbm, tok_v)

        def fill(r, carry):
            for j in range(d // 16):
                sl = pl.ds(16 * j, 16)
                tile_v[r, sl] = tok_v[0, sl]
            return carry

        lax.fori_loop(0, tile_v.shape[0], fill, 0)
        for b in range(_G // tile_v.shape[0]):
            pltpu.sync_copy(tile_v, out_hbm.at[pl.ds(tile_v.shape[0] * b, tile_v.shape[0])])

    n_mine = (n_chunks - wid + _NW - 1) // _NW

    def chunk_body(t, carry):
        start = (wid + _NW * t) * _C
        pltpu.sync_copy(deg_hbm.at[pl.ds(start, _C)], idx_v)
        pltpu.async_copy(tab_hbm.at[idx_v], rows_v, sem).wait()
        pltpu.sync_copy(x_hbm.at[pl.ds(start, _C)], x_v)

        def add_row(r, inner):
            for j in range(d // 16):
                sl = pl.ds(16 * j, 16)
                plsc.addupdate(rows_v.at[r, sl], x_v[r, sl])
            return inner

        lax.fori_loop(0, _C, add_row, 0)
        pltpu.sync_copy(rows_v, out_hbm.at[pl.ds(_G + start, _C)])
        return carry

    lax.fori_loop(0, n_mine, chunk_body, 0)


def kernel(x, out_degree, graph_ids, num_total_graphs, out_degree_table, graph_token):
    n, d = x.shape

    sc_call = pl.kernel(
        _sc_body,
        out_type=jax.ShapeDtypeStruct((_G + n, d), x.dtype),
        mesh=plsc.VectorSubcoreMesh(core_axis_name="c", subcore_axis_name="s"),
        scratch_types=[
            pltpu.VMEM((_C,), jnp.int32),
            pltpu.VMEM((_C, d), jnp.float32),
            pltpu.VMEM((_C, d), jnp.float32),
            pltpu.VMEM((1, d), jnp.float32),
            pltpu.VMEM((32, d), jnp.float32),
            pltpu.SemaphoreType.DMA,
        ],
    )
    feat = sc_call(x, out_degree, out_degree_table, graph_token)

    delta = (jnp.asarray(num_total_graphs) - _G).astype(graph_ids.dtype)
    tok_ids = jnp.arange(_G, dtype=graph_ids.dtype) + delta
    new_ids = jnp.concatenate([tok_ids, graph_ids], axis=0)
    return (feat, new_ids)


# speedup vs baseline: 1.2655x; 1.2655x over previous
import jax
import jax.numpy as jnp
from jax import lax
from jax.experimental import pallas as pl
from jax.experimental.pallas import tpu as pltpu
from jax.experimental.pallas import tpu_sc as plsc

_G = 256
_C = 40
_NW = 32


def _sc_body(x_hbm, deg_hbm, tab_hbm, tok_hbm, out_hbm,
             idx0, idx1, rows0, rows1, xb0, xb1, tok_v, tile_v,
             s_i0, s_i1, s_g0, s_g1, s_x0, s_x1, s_o0, s_o1):
    cid = lax.axis_index("c")
    sid = lax.axis_index("s")
    wid = sid * 2 + cid
    n, d = x_hbm.shape
    n_chunks = n // _C

    @pl.when(wid == _NW - 1)
    def _():
        pltpu.sync_copy(tok_hbm, tok_v)

        def fill(r, carry):
            for j in range(d // 16):
                sl = pl.ds(16 * j, 16)
                tile_v[r, sl] = tok_v[0, sl]
            return carry

        lax.fori_loop(0, tile_v.shape[0], fill, 0)
        for b in range(_G // tile_v.shape[0]):
            pltpu.sync_copy(tile_v, out_hbm.at[pl.ds(tile_v.shape[0] * b, tile_v.shape[0])])

    bufs = ((idx0, rows0, xb0, s_i0, s_g0, s_x0, s_o0),
            (idx1, rows1, xb1, s_i1, s_g1, s_x1, s_o1))

    def deg_sl(kid):
        return deg_hbm.at[pl.ds(kid * _C, _C)]

    def x_sl(kid):
        return x_hbm.at[pl.ds(kid * _C, _C)]

    def out_sl(kid):
        return out_hbm.at[pl.ds(_G + kid * _C, _C)]

    pltpu.async_copy(deg_sl(wid), idx0, s_i0)
    pltpu.make_async_copy(deg_sl(wid), idx0, s_i0).wait()
    pltpu.async_copy(tab_hbm.at[idx0], rows0, s_g0)
    pltpu.async_copy(x_sl(wid), xb0, s_x0)

    @pl.when(wid + _NW < n_chunks)
    def _():
        pltpu.async_copy(deg_sl(wid + _NW), idx1, s_i1)

    def half(k, p):
        idx_b, rows_b, x_b, s_i, s_g, s_x, s_o = bufs[p]
        idx_q, rows_q, x_q, s_iq, s_gq, s_xq, s_oq = bufs[1 - p]
        kid = wid + _NW * k

        @pl.when(kid < n_chunks)
        def _():
            pltpu.make_async_copy(tab_hbm.at[idx_b], rows_b, s_g).wait()
            pltpu.make_async_copy(x_sl(kid), x_b, s_x).wait()

            @pl.when(kid + 2 * _NW < n_chunks)
            def _():
                pltpu.async_copy(deg_sl(kid + 2 * _NW), idx_b, s_i)

            def add_row(r, carry):
                for j in range(d // 16):
                    sl = pl.ds(16 * j, 16)
                    plsc.addupdate(rows_b.at[r, sl], x_b[r, sl])
                return carry

            lax.fori_loop(0, _C, add_row, 0)
            pltpu.async_copy(rows_b, out_sl(kid), s_o)

            @pl.when(kid + _NW < n_chunks)
            def _():
                pltpu.make_async_copy(deg_sl(kid + _NW), idx_q, s_iq).wait()

                @pl.when(kid - _NW >= 0)
                def _():
                    pltpu.make_async_copy(rows_q, out_sl(kid - _NW), s_oq).wait()

                pltpu.async_copy(tab_hbm.at[idx_q], rows_q, s_gq)
                pltpu.async_copy(x_sl(kid + _NW), x_q, s_xq)

    n_mine = (n_chunks - wid + _NW - 1) // _NW

    def pair(g, carry):
        half(2 * g, 0)
        half(2 * g + 1, 1)
        return carry

    lax.fori_loop(0, (n_mine + 1) // 2, pair, 0)

    k_last = n_mine - 1

    def drain(k, p):
        idx_b, rows_b, x_b, s_i, s_g, s_x, s_o = bufs[p]
        kid = wid + _NW * k

        @pl.when((k >= 0) & (k % 2 == p))
        def _():
            pltpu.make_async_copy(rows_b, out_sl(kid), s_o).wait()

    for p in (0, 1):
        drain(k_last, p)
        drain(k_last - 1, p)


def kernel(x, out_degree, graph_ids, num_total_graphs, out_degree_table, graph_token):
    n, d = x.shape

    sc_call = pl.kernel(
        _sc_body,
        out_type=jax.ShapeDtypeStruct((_G + n, d), x.dtype),
        mesh=plsc.VectorSubcoreMesh(core_axis_name="c", subcore_axis_name="s"),
        scratch_types=[
            pltpu.VMEM((_C,), jnp.int32),
            pltpu.VMEM((_C,), jnp.int32),
            pltpu.VMEM((_C, d), jnp.float32),
            pltpu.VMEM((_C, d), jnp.float32),
            pltpu.VMEM((_C, d), jnp.float32),
            pltpu.VMEM((_C, d), jnp.float32),
            pltpu.VMEM((1, d), jnp.float32),
            pltpu.VMEM((32, d), jnp.float32),
            pltpu.SemaphoreType.DMA,
            pltpu.SemaphoreType.DMA,
            pltpu.SemaphoreType.DMA,
            pltpu.SemaphoreType.DMA,
            pltpu.SemaphoreType.DMA,
            pltpu.SemaphoreType.DMA,
            pltpu.SemaphoreType.DMA,
            pltpu.SemaphoreType.DMA,
        ],
    )
    feat = sc_call(x, out_degree, out_degree_table, graph_token)

    delta = (jnp.asarray(num_total_graphs) - _G).astype(graph_ids.dtype)
    tok_ids = jnp.arange(_G, dtype=graph_ids.dtype) + delta
    new_ids = jnp.concatenate([tok_ids, graph_ids], axis=0)
    return (feat, new_ids)
